# Initial kernel scaffold; baseline (speedup 1.0000x reference)
import jax
import jax.numpy as jnp
from jax.experimental import pallas as pl


def kernel(h, edge_index, e, params):
    raise NotImplementedError("write your pallas kernel here")


# speedup vs baseline: 11.6511x; 11.6511x over previous
import functools

import jax
import jax.numpy as jnp
from jax import lax
from jax.experimental import pallas as pl
from jax.experimental.pallas import tpu as pltpu
from jax.experimental.pallas import tpu_sc as plsc

_N = 10000
_B = 100
_NPG = 100
_E = 160000
_D = 256
_K = 50
_NCLS = 2

_SC_CORES = 2
_SC_TILES = 16
_EPC = _E // _SC_CORES
_EPT = _EPC // _SC_TILES
_SEG = _B * _NPG * _NPG // _SC_CORES
_CH = 128
_NCH = (_EPT + _CH - 1) // _CH
_NG16 = _NCH * (_CH // 16)
_ZCH = 31248


def _build_adj(src, dst):
    mesh = plsc.VectorSubcoreMesh(core_axis_name="c", subcore_axis_name="s")

    @functools.partial(
        pl.kernel,
        out_type=jax.ShapeDtypeStruct((_B * _NPG * _NPG,), jnp.float32),
        mesh=mesh,
        scratch_types=[
            pltpu.VMEM((_NG16 * 16,), jnp.int32),
            pltpu.VMEM((_NG16 * 16,), jnp.int32),
            pltpu.VMEM((_NCH, _CH), jnp.int32),
            pltpu.VMEM((_NCH, _CH), jnp.float32),
            pltpu.VMEM((_ZCH,), jnp.float32),
            pltpu.VMEM_SHARED((_SEG,), jnp.float32),
        ],
    )
    def adj_kernel(src_hbm, dst_hbm, out_hbm, src_v, dst_v, idx_v, val_v,
                   zero_v, acc_s):
        cid = lax.axis_index("c")
        tid = lax.axis_index("s")

        def zbody(i, _):
            zero_v[pl.ds(i * 16, 16)] = jnp.zeros((16,), jnp.float32)
            return 0

        lax.fori_loop(0, _ZCH // 16, zbody, 0)
        pltpu.sync_copy(zero_v, acc_s.at[pl.ds(tid * _ZCH, _ZCH)])

        @pl.when(tid == 0)
        def _():
            rem = _SEG - _SC_TILES * _ZCH
            pltpu.sync_copy(zero_v.at[pl.ds(0, rem)],
                            acc_s.at[pl.ds(_SC_TILES * _ZCH, rem)])

        base = cid * _EPC + tid * _EPT
        pltpu.sync_copy(src_hbm.at[pl.ds(base, _EPT)],
                        src_v.at[pl.ds(0, _EPT)])
        pltpu.sync_copy(dst_hbm.at[pl.ds(base, _EPT)],
                        dst_v.at[pl.ds(0, _EPT)])

        seg_base = cid * _SEG

        def cbody(g, _):
            sv = src_v[pl.ds(g * 16, 16)]
            dv = dst_v[pl.ds(g * 16, 16)]
            lane = g * 16 + lax.iota(jnp.int32, 16)
            valid = lane < _EPT
            idx = sv * _NPG + lax.rem(dv, _NPG) - seg_base
            idx = jnp.where(valid, idx, 0)
            val = jnp.where(valid, jnp.float32(1.0), jnp.float32(0.0))
            row = g // (_CH // 16)
            col = lax.rem(g, _CH // 16) * 16
            idx_v[row, pl.ds(col, 16)] = idx
            val_v[row, pl.ds(col, 16)] = val
            return 0

        lax.fori_loop(0, _NG16, cbody, 0)

        plsc.subcore_barrier()

        for j in range(_NCH):
            pltpu.sync_copy(val_v.at[j], acc_s.at[idx_v.at[j]], add=True)

        plsc.subcore_barrier()

        out_base = cid * _SEG + tid * _ZCH
        pltpu.sync_copy(acc_s.at[pl.ds(tid * _ZCH, _ZCH)], zero_v)
        pltpu.sync_copy(zero_v, out_hbm.at[pl.ds(out_base, _ZCH)])

        @pl.when(tid == 0)
        def _():
            rem = _SEG - _SC_TILES * _ZCH
            pltpu.sync_copy(acc_s.at[pl.ds(_SC_TILES * _ZCH, rem)],
                            zero_v.at[pl.ds(0, rem)])
            pltpu.sync_copy(zero_v.at[pl.ds(0, rem)],
                            out_hbm.at[pl.ds(cid * _SEG + _SC_TILES * _ZCH,
                                             rem)])

    return adj_kernel(src, dst)


_GB = 20
_NSTEP = _B // _GB
_RB = _GB * _NPG


def _dot(a, b):
    return jnp.dot(a, b, preferred_element_type=jnp.float32)


def _dot_t(a, b):
    return lax.dot_general(a, b, (((0,), (0,)), ((), ())),
                           preferred_element_type=jnp.float32)


def _embed(h, w, b):
    def body(h_ref, w_ref, b_ref, o_ref):
        o_ref[...] = _dot(h_ref[...], w_ref[...]) + b_ref[...]

    return pl.pallas_call(
        body,
        grid=(_NSTEP,),
        in_specs=[
            pl.BlockSpec((_RB, _D), lambda i: (i, 0)),
            pl.BlockSpec((_D, _D), lambda i: (0, 0)),
            pl.BlockSpec((1, _D), lambda i: (0, 0)),
        ],
        out_specs=pl.BlockSpec((_RB, _D), lambda i: (i, 0)),
        out_shape=jax.ShapeDtypeStruct((_N, _D), jnp.float32),
    )(h, w, b)


def _sage_mm(h3, adj, ws, wn, bs, with_stats):
    def body(h_ref, adj_ref, ws_ref, wn_ref, bs_ref, hn_ref, *rest):
        if with_stats:
            sums_ref, acc_ref = rest
            step = pl.program_id(0)

            @pl.when(step == 0)
            def _():
                acc_ref[...] = jnp.zeros_like(acc_ref)

        for g in range(_GB):
            a = adj_ref[g]
            hg = h_ref[g]
            inv_indeg = 1.0 / jnp.maximum(
                jnp.sum(a, axis=0, keepdims=True), 1.0)
            agg = _dot_t(a * inv_indeg, hg)
            hn = _dot(hg, ws_ref[...]) + _dot(agg, wn_ref[...]) + bs_ref[...]
            if with_stats:
                hn_ref[g] = hn
                acc_ref[0:1, :] += jnp.sum(hn, axis=0, keepdims=True)
                acc_ref[1:2, :] += jnp.sum(hn * hn, axis=0, keepdims=True)
            else:
                hn_ref[g] = hn + hg

        if with_stats:
            @pl.when(step == pl.num_programs(0) - 1)
            def _():
                sums_ref[...] = acc_ref[...]

    out_shapes = [jax.ShapeDtypeStruct((_B, _NPG, _D), jnp.float32)]
    out_specs = [pl.BlockSpec((_GB, _NPG, _D), lambda i: (i, 0, 0))]
    scratch = []
    if with_stats:
        out_shapes.append(jax.ShapeDtypeStruct((2, _D), jnp.float32))
        out_specs.append(pl.BlockSpec((2, _D), lambda i: (0, 0)))
        scratch.append(pltpu.VMEM((2, _D), jnp.float32))

    res = pl.pallas_call(
        body,
        grid=(_NSTEP,),
        in_specs=[
            pl.BlockSpec((_GB, _NPG, _D), lambda i: (i, 0, 0)),
            pl.BlockSpec((_GB, _NPG, _NPG), lambda i: (i, 0, 0)),
            pl.BlockSpec((_D, _D), lambda i: (0, 0)),
            pl.BlockSpec((_D, _D), lambda i: (0, 0)),
            pl.BlockSpec((1, _D), lambda i: (0, 0)),
        ],
        out_specs=out_specs,
        out_shape=out_shapes,
        scratch_shapes=scratch,
    )(h3, adj, ws, wn, bs)
    return res if with_stats else res[0]


def _norm_res(hn, h, sums, g, b):
    def body(hn_ref, h_ref, sums_ref, g_ref, b_ref, o_ref):
        mu = sums_ref[0:1, :] * (1.0 / _N)
        ex2 = sums_ref[1:2, :] * (1.0 / _N)
        var = ex2 - mu * mu
        rstd = lax.rsqrt(var + 1e-5)
        y = (hn_ref[...] - mu) * (rstd * g_ref[...]) + b_ref[...]
        o_ref[...] = jnp.maximum(y, 0.0) + h_ref[...]

    return pl.pallas_call(
        body,
        grid=(_NSTEP,),
        in_specs=[
            pl.BlockSpec((_RB, _D), lambda i: (i, 0)),
            pl.BlockSpec((_RB, _D), lambda i: (i, 0)),
            pl.BlockSpec((2, _D), lambda i: (0, 0)),
            pl.BlockSpec((1, _D), lambda i: (0, 0)),
            pl.BlockSpec((1, _D), lambda i: (0, 0)),
        ],
        out_specs=pl.BlockSpec((_RB, _D), lambda i: (i, 0)),
        out_shape=jax.ShapeDtypeStruct((_N, _D), jnp.float32),
    )(hn, h, sums, g, b)


def _pool(h3, adj, wa_n, wa_s):
    def body(h_ref, adj_ref, wan_ref, was_ref, hp_ref, ap_ref):
        for g in range(_GB):
            a = adj_ref[g]
            hg = h_ref[g]
            adjn = a / jnp.maximum(jnp.sum(a, axis=1, keepdims=True), 1.0)
            t1 = _dot(adjn, hg)
            logits = _dot(t1, wan_ref[...]) + _dot(hg, was_ref[...])
            m = jnp.max(logits, axis=1, keepdims=True)
            p = jnp.exp(logits - m)
            s = p / jnp.sum(p, axis=1, keepdims=True)
            hp_ref[g] = _dot_t(s, hg)
            adjp = _dot_t(s, _dot(a, s))
            ap_ref[g] = adjp / jnp.maximum(
                jnp.sum(adjp, axis=1, keepdims=True), 1.0)

    return pl.pallas_call(
        body,
        grid=(_NSTEP,),
        in_specs=[
            pl.BlockSpec((_GB, _NPG, _D), lambda i: (i, 0, 0)),
            pl.BlockSpec((_GB, _NPG, _NPG), lambda i: (i, 0, 0)),
            pl.BlockSpec((_D, _K), lambda i: (0, 0)),
            pl.BlockSpec((_D, _K), lambda i: (0, 0)),
        ],
        out_specs=[
            pl.BlockSpec((_GB, _K, _D), lambda i: (i, 0, 0)),
            pl.BlockSpec((_GB, _K, _K), lambda i: (i, 0, 0)),
        ],
        out_shape=[
            jax.ShapeDtypeStruct((_B, _K, _D), jnp.float32),
            jax.ShapeDtypeStruct((_B, _K, _K), jnp.float32),
        ],
    )(h3, adj, wa_n, wa_s)


def _dsage(hp3, ap, ws0, wn0, ws1, wn1, ws2, wn2):
    def body(hp_ref, ap_ref, ws0_ref, wn0_ref, ws1_ref, wn1_ref, ws2_ref,
             wn2_ref, o_ref):
        wpairs = [(ws0_ref, wn0_ref), (ws1_ref, wn1_ref), (ws2_ref, wn2_ref)]
        for g in range(_GB):
            x = hp_ref[g]
            apg = ap_ref[g]
            for i, (ws_ref, wn_ref) in enumerate(wpairs):
                xn = _dot(apg, _dot(x, wn_ref[...])) + _dot(x, ws_ref[...])
                if i < 2:
                    xn = jnp.maximum(xn, 0.0)
                x = xn + x
            o_ref[g] = x

    wspec = pl.BlockSpec((_D, _D), lambda i: (0, 0))
    return pl.pallas_call(
        body,
        grid=(_NSTEP,),
        in_specs=[
            pl.BlockSpec((_GB, _K, _D), lambda i: (i, 0, 0)),
            pl.BlockSpec((_GB, _K, _K), lambda i: (i, 0, 0)),
            wspec, wspec, wspec, wspec, wspec, wspec,
        ],
        out_specs=pl.BlockSpec((_GB, _K, _D), lambda i: (i, 0, 0)),
        out_shape=jax.ShapeDtypeStruct((_B, _K, _D), jnp.float32),
    )(hp3, ap, ws0, wn0, ws1, wn1, ws2, wn2)


def _readout(x3, wp, bp):
    def body(x_ref, wp_ref, bp_ref, o_ref):
        r = jnp.sum(x_ref[...], axis=1)
        o_ref[...] = _dot(r, wp_ref[...]) + bp_ref[...]

    return pl.pallas_call(
        body,
        in_specs=[
            pl.BlockSpec((_B, _K, _D), lambda: (0, 0, 0)),
            pl.BlockSpec((_D, _NCLS), lambda: (0, 0)),
            pl.BlockSpec((1, _NCLS), lambda: (0, 0)),
        ],
        out_specs=pl.BlockSpec((_B, _NCLS), lambda: (0, 0)),
        out_shape=jax.ShapeDtypeStruct((_B, _NCLS), jnp.float32),
    )(x3, wp, bp)


def kernel(h, edge_index, e, params):
    src = edge_index[0]
    dst = edge_index[1]

    adj_flat = _build_adj(src, dst)
    adj = adj_flat.reshape(_B, _NPG, _NPG)

    h0 = _embed(h, params['W_emb'], params['b_emb'].reshape(1, _D))
    h3 = h0.reshape(_B, _NPG, _D)

    for i, lyr in enumerate(params['sage']):
        if i < 2:
            hn3, sums = _sage_mm(h3, adj, lyr['Ws'], lyr['Wn'],
                                 lyr['bs'].reshape(1, _D), with_stats=True)
            hflat = _norm_res(hn3.reshape(_N, _D), h3.reshape(_N, _D), sums,
                              lyr['bn_g'].reshape(1, _D),
                              lyr['bn_b'].reshape(1, _D))
            h3 = hflat.reshape(_B, _NPG, _D)
        else:
            h3 = _sage_mm(h3, adj, lyr['Ws'], lyr['Wn'],
                          lyr['bs'].reshape(1, _D), with_stats=False)

    hp3, ap = _pool(h3, adj, params['Wa_n'], params['Wa_s'])
    d = params['dsage']
    x3 = _dsage(hp3, ap, d[0]['Ws'], d[0]['Wn'], d[1]['Ws'], d[1]['Wn'],
                d[2]['Ws'], d[2]['Wn'])
    return _readout(x3, params['W_pred'],
                    params['b_pred'].reshape(1, _NCLS))


# speedup vs baseline: 14.3989x; 1.2358x over previous
import functools

import jax
import jax.numpy as jnp
from jax import lax
from jax.experimental import pallas as pl
from jax.experimental.pallas import tpu as pltpu
from jax.experimental.pallas import tpu_sc as plsc

_N = 10000
_B = 100
_NPG = 100
_E = 160000
_D = 256
_K = 50
_NCLS = 2

_SC_CORES = 2
_SC_TILES = 16
_EPC = _E // _SC_CORES
_EPT = _EPC // _SC_TILES
_SEG = _B * _NPG * _NPG // _SC_CORES
_CH = 128
_NCH = (_EPT + _CH - 1) // _CH
_NG16 = _NCH * (_CH // 16)
_ZCH = 31248


def _build_adj(src, dst):
    mesh = plsc.VectorSubcoreMesh(core_axis_name="c", subcore_axis_name="s")

    @functools.partial(
        pl.kernel,
        out_type=jax.ShapeDtypeStruct((_B * _NPG * _NPG,), jnp.float32),
        mesh=mesh,
        scratch_types=[
            pltpu.VMEM((_NG16 * 16,), jnp.int32),
            pltpu.VMEM((_NG16 * 16,), jnp.int32),
            pltpu.VMEM((_NCH, _CH), jnp.int32),
            pltpu.VMEM((_NCH, _CH), jnp.float32),
            pltpu.VMEM((_ZCH,), jnp.float32),
            pltpu.VMEM_SHARED((_SEG,), jnp.float32),
            pltpu.SemaphoreType.DMA,
        ],
    )
    def adj_kernel(src_hbm, dst_hbm, out_hbm, src_v, dst_v, idx_v, val_v,
                   zero_v, acc_s, scat_sem):
        cid = lax.axis_index("c")
        tid = lax.axis_index("s")

        def zbody(i, _):
            zero_v[pl.ds(i * 16, 16)] = jnp.zeros((16,), jnp.float32)
            return 0

        lax.fori_loop(0, _ZCH // 16, zbody, 0)
        pltpu.sync_copy(zero_v, acc_s.at[pl.ds(tid * _ZCH, _ZCH)])

        @pl.when(tid == 0)
        def _():
            rem = _SEG - _SC_TILES * _ZCH
            pltpu.sync_copy(zero_v.at[pl.ds(0, rem)],
                            acc_s.at[pl.ds(_SC_TILES * _ZCH, rem)])

        base = cid * _EPC + tid * _EPT
        pltpu.sync_copy(src_hbm.at[pl.ds(base, _EPT)],
                        src_v.at[pl.ds(0, _EPT)])
        pltpu.sync_copy(dst_hbm.at[pl.ds(base, _EPT)],
                        dst_v.at[pl.ds(0, _EPT)])

        seg_base = cid * _SEG

        def cbody(g, _):
            sv = src_v[pl.ds(g * 16, 16)]
            dv = dst_v[pl.ds(g * 16, 16)]
            lane = g * 16 + lax.iota(jnp.int32, 16)
            valid = lane < _EPT
            idx = sv * _NPG + lax.rem(dv, _NPG) - seg_base
            idx = jnp.where(valid, idx, 0)
            val = jnp.where(valid, jnp.float32(1.0), jnp.float32(0.0))
            row = g // (_CH // 16)
            col = lax.rem(g, _CH // 16) * 16
            idx_v[row, pl.ds(col, 16)] = idx
            val_v[row, pl.ds(col, 16)] = val
            return 0

        lax.fori_loop(0, _NG16, cbody, 0)

        plsc.subcore_barrier()

        copies = [
            pltpu.async_copy(val_v.at[j], acc_s.at[idx_v.at[j]], scat_sem,
                             add=True)
            for j in range(_NCH)
        ]
        for c in copies:
            c.wait()

        plsc.subcore_barrier()

        out_base = cid * _SEG + tid * _ZCH
        pltpu.sync_copy(acc_s.at[pl.ds(tid * _ZCH, _ZCH)], zero_v)
        pltpu.sync_copy(zero_v, out_hbm.at[pl.ds(out_base, _ZCH)])

        @pl.when(tid == 0)
        def _():
            rem = _SEG - _SC_TILES * _ZCH
            pltpu.sync_copy(acc_s.at[pl.ds(_SC_TILES * _ZCH, rem)],
                            zero_v.at[pl.ds(0, rem)])
            pltpu.sync_copy(zero_v.at[pl.ds(0, rem)],
                            out_hbm.at[pl.ds(cid * _SEG + _SC_TILES * _ZCH,
                                             rem)])

    return adj_kernel(src, dst)


_GB = 20
_NSTEP = _B // _GB
_RB = _GB * _NPG


def _dot(a, b):
    return jnp.dot(a, b, preferred_element_type=jnp.float32)


def _dot_t(a, b):
    return lax.dot_general(a, b, (((0,), (0,)), ((), ())),
                           preferred_element_type=jnp.float32)


def _embed(h, w, b):
    def body(h_ref, w_ref, b_ref, o_ref):
        o_ref[...] = _dot(h_ref[...], w_ref[...]) + b_ref[...]

    return pl.pallas_call(
        body,
        grid=(_NSTEP,),
        in_specs=[
            pl.BlockSpec((_RB, _D), lambda i: (i, 0)),
            pl.BlockSpec((_D, _D), lambda i: (0, 0)),
            pl.BlockSpec((1, _D), lambda i: (0, 0)),
        ],
        out_specs=pl.BlockSpec((_RB, _D), lambda i: (i, 0)),
        out_shape=jax.ShapeDtypeStruct((_N, _D), jnp.float32),
    )(h, w, b)


def _sage0(h3, adj, ws, wn, bs):
    def body(h_ref, adj_ref, ws_ref, wn_ref, bs_ref, hn_ref, sums_ref,
             acc_ref):
        step = pl.program_id(0)

        @pl.when(step == 0)
        def _():
            acc_ref[...] = jnp.zeros_like(acc_ref)

        for g in range(_GB):
            a = adj_ref[g]
            hg = h_ref[g]
            inv_indeg = 1.0 / jnp.maximum(
                jnp.sum(a, axis=0, keepdims=True), 1.0)
            agg = _dot_t(a * inv_indeg, hg)
            hn = _dot(hg, ws_ref[...]) + _dot(agg, wn_ref[...]) + bs_ref[...]
            hn_ref[g] = hn
            acc_ref[0:1, :] += jnp.sum(hn, axis=0, keepdims=True)
            acc_ref[1:2, :] += jnp.sum(hn * hn, axis=0, keepdims=True)

        @pl.when(step == pl.num_programs(0) - 1)
        def _():
            sums_ref[...] = acc_ref[...]

    big = pl.BlockSpec((_GB, _NPG, _D), lambda i: (i, 0, 0))
    return pl.pallas_call(
        body,
        grid=(_NSTEP,),
        in_specs=[
            big,
            pl.BlockSpec((_GB, _NPG, _NPG), lambda i: (i, 0, 0)),
            pl.BlockSpec((_D, _D), lambda i: (0, 0)),
            pl.BlockSpec((_D, _D), lambda i: (0, 0)),
            pl.BlockSpec((1, _D), lambda i: (0, 0)),
        ],
        out_specs=[big, pl.BlockSpec((2, _D), lambda i: (0, 0))],
        out_shape=[
            jax.ShapeDtypeStruct((_B, _NPG, _D), jnp.float32),
            jax.ShapeDtypeStruct((2, _D), jnp.float32),
        ],
        scratch_shapes=[pltpu.VMEM((2, _D), jnp.float32)],
    )(h3, adj, ws, wn, bs)


def _sage_next(hn_prev, h_prev, sums_prev, bng, bnb, adj, ws, wn, bs,
               with_stats):
    def body(hnp_ref, hp_ref, sums_ref, bng_ref, bnb_ref, adj_ref, ws_ref,
             wn_ref, bs_ref, *outs):
        mu = sums_ref[0:1, :] * (1.0 / _N)
        var = sums_ref[1:2, :] * (1.0 / _N) - mu * mu
        scale = lax.rsqrt(var + 1e-5) * bng_ref[...]

        if with_stats:
            h_ref, hn_ref, souts_ref, acc_ref = outs
            step = pl.program_id(0)

            @pl.when(step == 0)
            def _():
                acc_ref[...] = jnp.zeros_like(acc_ref)
        else:
            h_ref, = outs

        for g in range(_GB):
            y = (hnp_ref[g] - mu) * scale + bnb_ref[...]
            hg = jnp.maximum(y, 0.0) + hp_ref[g]
            a = adj_ref[g]
            inv_indeg = 1.0 / jnp.maximum(
                jnp.sum(a, axis=0, keepdims=True), 1.0)
            agg = _dot_t(a * inv_indeg, hg)
            hn = _dot(hg, ws_ref[...]) + _dot(agg, wn_ref[...]) + bs_ref[...]
            if with_stats:
                h_ref[g] = hg
                hn_ref[g] = hn
                acc_ref[0:1, :] += jnp.sum(hn, axis=0, keepdims=True)
                acc_ref[1:2, :] += jnp.sum(hn * hn, axis=0, keepdims=True)
            else:
                h_ref[g] = hn + hg

        if with_stats:
            @pl.when(step == pl.num_programs(0) - 1)
            def _():
                souts_ref[...] = acc_ref[...]

    big = pl.BlockSpec((_GB, _NPG, _D), lambda i: (i, 0, 0))
    row = pl.BlockSpec((1, _D), lambda i: (0, 0))
    stat = pl.BlockSpec((2, _D), lambda i: (0, 0))
    out_specs = [big]
    out_shapes = [jax.ShapeDtypeStruct((_B, _NPG, _D), jnp.float32)]
    scratch = []
    if with_stats:
        out_specs += [big, stat]
        out_shapes += [jax.ShapeDtypeStruct((_B, _NPG, _D), jnp.float32),
                       jax.ShapeDtypeStruct((2, _D), jnp.float32)]
        scratch.append(pltpu.VMEM((2, _D), jnp.float32))

    res = pl.pallas_call(
        body,
        grid=(_NSTEP,),
        in_specs=[
            big, big, stat, row, row,
            pl.BlockSpec((_GB, _NPG, _NPG), lambda i: (i, 0, 0)),
            pl.BlockSpec((_D, _D), lambda i: (0, 0)),
            pl.BlockSpec((_D, _D), lambda i: (0, 0)),
            row,
        ],
        out_specs=out_specs,
        out_shape=out_shapes,
        scratch_shapes=scratch,
    )(hn_prev, h_prev, sums_prev, bng, bnb, adj, ws, wn, bs)
    return res if with_stats else res[0]


def _pool_dsage(h3, adj, wa_n, wa_s, ws0, wn0, ws1, wn1, ws2, wn2):
    def body(h_ref, adj_ref, wan_ref, was_ref, ws0_ref, wn0_ref, ws1_ref,
             wn1_ref, ws2_ref, wn2_ref, o_ref):
        wpairs = [(ws0_ref, wn0_ref), (ws1_ref, wn1_ref), (ws2_ref, wn2_ref)]
        for g in range(_GB):
            a = adj_ref[g]
            hg = h_ref[g]
            adjn = a / jnp.maximum(jnp.sum(a, axis=1, keepdims=True), 1.0)
            t1 = _dot(adjn, hg)
            logits = _dot(t1, wan_ref[...]) + _dot(hg, was_ref[...])
            m = jnp.max(logits, axis=1, keepdims=True)
            pexp = jnp.exp(logits - m)
            s = pexp / jnp.sum(pexp, axis=1, keepdims=True)
            x = _dot_t(s, hg)
            adjp = _dot_t(s, _dot(a, s))
            apg = adjp / jnp.maximum(jnp.sum(adjp, axis=1, keepdims=True),
                                     1.0)
            for i, (ws_ref, wn_ref) in enumerate(wpairs):
                xn = _dot(apg, _dot(x, wn_ref[...])) + _dot(x, ws_ref[...])
                if i < 2:
                    xn = jnp.maximum(xn, 0.0)
                x = xn + x
            o_ref[g] = x

    wspec = pl.BlockSpec((_D, _D), lambda i: (0, 0))
    return pl.pallas_call(
        body,
        grid=(_NSTEP,),
        in_specs=[
            pl.BlockSpec((_GB, _NPG, _D), lambda i: (i, 0, 0)),
            pl.BlockSpec((_GB, _NPG, _NPG), lambda i: (i, 0, 0)),
            pl.BlockSpec((_D, _K), lambda i: (0, 0)),
            pl.BlockSpec((_D, _K), lambda i: (0, 0)),
            wspec, wspec, wspec, wspec, wspec, wspec,
        ],
        out_specs=pl.BlockSpec((_GB, _K, _D), lambda i: (i, 0, 0)),
        out_shape=jax.ShapeDtypeStruct((_B, _K, _D), jnp.float32),
    )(h3, adj, wa_n, wa_s, ws0, wn0, ws1, wn1, ws2, wn2)


def _readout(x3, wp, bp):
    def body(x_ref, wp_ref, bp_ref, o_ref):
        r = jnp.sum(x_ref[...], axis=1)
        o_ref[...] = _dot(r, wp_ref[...]) + bp_ref[...]

    return pl.pallas_call(
        body,
        in_specs=[
            pl.BlockSpec((_B, _K, _D), lambda: (0, 0, 0)),
            pl.BlockSpec((_D, _NCLS), lambda: (0, 0)),
            pl.BlockSpec((1, _NCLS), lambda: (0, 0)),
        ],
        out_specs=pl.BlockSpec((_B, _NCLS), lambda: (0, 0)),
        out_shape=jax.ShapeDtypeStruct((_B, _NCLS), jnp.float32),
    )(x3, wp, bp)


def kernel(h, edge_index, e, params):
    src = edge_index[0]
    dst = edge_index[1]

    adj_flat = _build_adj(src, dst)
    adj = adj_flat.reshape(_B, _NPG, _NPG)

    h0 = _embed(h, params['W_emb'], params['b_emb'].reshape(1, _D))
    h3 = h0.reshape(_B, _NPG, _D)

    sage = params['sage']
    hn0, sums0 = _sage0(h3, adj, sage[0]['Ws'], sage[0]['Wn'],
                        sage[0]['bs'].reshape(1, _D))
    h1, hn1, sums1 = _sage_next(
        hn0, h3, sums0, sage[0]['bn_g'].reshape(1, _D),
        sage[0]['bn_b'].reshape(1, _D), adj, sage[1]['Ws'], sage[1]['Wn'],
        sage[1]['bs'].reshape(1, _D), with_stats=True)
    h_final = _sage_next(
        hn1, h1, sums1, sage[1]['bn_g'].reshape(1, _D),
        sage[1]['bn_b'].reshape(1, _D), adj, sage[2]['Ws'], sage[2]['Wn'],
        sage[2]['bs'].reshape(1, _D), with_stats=False)

    d = params['dsage']
    x3 = _pool_dsage(h_final, adj, params['Wa_n'], params['Wa_s'],
                     d[0]['Ws'], d[0]['Wn'], d[1]['Ws'], d[1]['Wn'],
                     d[2]['Ws'], d[2]['Wn'])
    return _readout(x3, params['W_pred'],
                    params['b_pred'].reshape(1, _NCLS))


# speedup vs baseline: 23.7604x; 1.6502x over previous
import functools

import jax
import jax.numpy as jnp
from jax import lax
from jax.experimental import pallas as pl
from jax.experimental.pallas import tpu as pltpu
from jax.experimental.pallas import tpu_sc as plsc

_N = 10000
_B = 100
_NPG = 100
_E = 160000
_D = 256
_K = 50
_NCLS = 2

_SC_CORES = 2
_SC_TILES = 16
_EPC = _E // _SC_CORES
_EPT = _EPC // _SC_TILES
_SEG = _B * _NPG * _NPG // _SC_CORES
_CH = 128
_NCH = (_EPT + _CH - 1) // _CH
_NG16 = _NCH * (_CH // 16)
_ZCH = 31248


def _build_adj(src, dst):
    mesh = plsc.VectorSubcoreMesh(core_axis_name="c", subcore_axis_name="s")

    @functools.partial(
        pl.kernel,
        out_type=jax.ShapeDtypeStruct((_B * _NPG * _NPG,), jnp.float32),
        mesh=mesh,
        scratch_types=[
            pltpu.VMEM((_NG16 * 16,), jnp.int32),
            pltpu.VMEM((_NG16 * 16,), jnp.int32),
            pltpu.VMEM((_NCH, _CH), jnp.int32),
            pltpu.VMEM((_NCH, _CH), jnp.float32),
            pltpu.VMEM((_ZCH,), jnp.float32),
            pltpu.VMEM_SHARED((_SEG,), jnp.float32),
            pltpu.SemaphoreType.DMA,
        ],
    )
    def adj_kernel(src_hbm, dst_hbm, out_hbm, src_v, dst_v, idx_v, val_v,
                   zero_v, acc_s, scat_sem):
        cid = lax.axis_index("c")
        tid = lax.axis_index("s")

        def zbody(i, _):
            zero_v[pl.ds(i * 16, 16)] = jnp.zeros((16,), jnp.float32)
            return 0

        lax.fori_loop(0, _ZCH // 16, zbody, 0)
        pltpu.sync_copy(zero_v, acc_s.at[pl.ds(tid * _ZCH, _ZCH)])

        @pl.when(tid == 0)
        def _():
            rem = _SEG - _SC_TILES * _ZCH
            pltpu.sync_copy(zero_v.at[pl.ds(0, rem)],
                            acc_s.at[pl.ds(_SC_TILES * _ZCH, rem)])

        base = cid * _EPC + tid * _EPT
        pltpu.sync_copy(src_hbm.at[pl.ds(base, _EPT)],
                        src_v.at[pl.ds(0, _EPT)])
        pltpu.sync_copy(dst_hbm.at[pl.ds(base, _EPT)],
                        dst_v.at[pl.ds(0, _EPT)])

        seg_base = cid * _SEG

        def cbody(g, _):
            sv = src_v[pl.ds(g * 16, 16)]
            dv = dst_v[pl.ds(g * 16, 16)]
            lane = g * 16 + lax.iota(jnp.int32, 16)
            valid = lane < _EPT
            idx = sv * _NPG + lax.rem(dv, _NPG) - seg_base
            idx = jnp.where(valid, idx, 0)
            val = jnp.where(valid, jnp.float32(1.0), jnp.float32(0.0))
            row = g // (_CH // 16)
            col = lax.rem(g, _CH // 16) * 16
            idx_v[row, pl.ds(col, 16)] = idx
            val_v[row, pl.ds(col, 16)] = val
            return 0

        lax.fori_loop(0, _NG16, cbody, 0)

        plsc.subcore_barrier()

        copies = [
            pltpu.async_copy(val_v.at[j], acc_s.at[idx_v.at[j]], scat_sem,
                             add=True)
            for j in range(_NCH)
        ]
        for c in copies:
            c.wait()

        plsc.subcore_barrier()

        out_base = cid * _SEG + tid * _ZCH
        pltpu.sync_copy(acc_s.at[pl.ds(tid * _ZCH, _ZCH)], zero_v)
        pltpu.sync_copy(zero_v, out_hbm.at[pl.ds(out_base, _ZCH)])

        @pl.when(tid == 0)
        def _():
            rem = _SEG - _SC_TILES * _ZCH
            pltpu.sync_copy(acc_s.at[pl.ds(_SC_TILES * _ZCH, rem)],
                            zero_v.at[pl.ds(0, rem)])
            pltpu.sync_copy(zero_v.at[pl.ds(0, rem)],
                            out_hbm.at[pl.ds(cid * _SEG + _SC_TILES * _ZCH,
                                             rem)])

    return adj_kernel(src, dst)


_GB = 20
_NSTEP = _B // _GB
_RB = _GB * _NPG
_KB = _GB * _K


def _dot(a, b):
    return jnp.dot(a, b, preferred_element_type=jnp.float32)


def _dot_t(a, b):
    return lax.dot_general(a, b, (((0,), (0,)), ((), ())),
                           preferred_element_type=jnp.float32)


def _agg_block(adj_ref, hb):
    aggs = []
    for g in range(_GB):
        a = adj_ref[g]
        inv_indeg = 1.0 / jnp.maximum(jnp.sum(a, axis=0, keepdims=True), 1.0)
        aggs.append(_dot_t(a * inv_indeg, hb[g * _NPG:(g + 1) * _NPG]))
    return jnp.concatenate(aggs, axis=0)


def _sage0(h, adj, wemb, bemb, ws, wn, bs):
    def body(h_ref, adj_ref, wemb_ref, bemb_ref, ws_ref, wn_ref, bs_ref,
             h0_ref, hn_ref, sums_ref, acc_ref):
        step = pl.program_id(0)

        @pl.when(step == 0)
        def _():
            acc_ref[...] = jnp.zeros_like(acc_ref)

        h0 = _dot(h_ref[...], wemb_ref[...]) + bemb_ref[...]
        agg = _agg_block(adj_ref, h0)
        hn = _dot(h0, ws_ref[...]) + _dot(agg, wn_ref[...]) + bs_ref[...]
        h0_ref[...] = h0
        hn_ref[...] = hn
        acc_ref[0:1, :] += jnp.sum(hn, axis=0, keepdims=True)
        acc_ref[1:2, :] += jnp.sum(hn * hn, axis=0, keepdims=True)

        @pl.when(step == pl.num_programs(0) - 1)
        def _():
            sums_ref[...] = acc_ref[...]

    big = pl.BlockSpec((_RB, _D), lambda i: (i, 0))
    row = pl.BlockSpec((1, _D), lambda i: (0, 0))
    return pl.pallas_call(
        body,
        grid=(_NSTEP,),
        in_specs=[
            big,
            pl.BlockSpec((_GB, _NPG, _NPG), lambda i: (i, 0, 0)),
            pl.BlockSpec((_D, _D), lambda i: (0, 0)),
            row,
            pl.BlockSpec((_D, _D), lambda i: (0, 0)),
            pl.BlockSpec((_D, _D), lambda i: (0, 0)),
            row,
        ],
        out_specs=[big, big, pl.BlockSpec((2, _D), lambda i: (0, 0))],
        out_shape=[
            jax.ShapeDtypeStruct((_N, _D), jnp.float32),
            jax.ShapeDtypeStruct((_N, _D), jnp.float32),
            jax.ShapeDtypeStruct((2, _D), jnp.float32),
        ],
        scratch_shapes=[pltpu.VMEM((2, _D), jnp.float32)],
    )(h, adj, wemb, bemb, ws, wn, bs)


def _sage_next(hn_prev, h_prev, sums_prev, bng, bnb, adj, ws, wn, bs,
               with_stats):
    def body(hnp_ref, hp_ref, sums_ref, bng_ref, bnb_ref, adj_ref, ws_ref,
             wn_ref, bs_ref, *outs):
        mu = sums_ref[0:1, :] * (1.0 / _N)
        var = sums_ref[1:2, :] * (1.0 / _N) - mu * mu
        scale = lax.rsqrt(var + 1e-5) * bng_ref[...]

        if with_stats:
            h_ref, hn_ref, souts_ref, acc_ref = outs
            step = pl.program_id(0)

            @pl.when(step == 0)
            def _():
                acc_ref[...] = jnp.zeros_like(acc_ref)
        else:
            h_ref, = outs

        y = (hnp_ref[...] - mu) * scale + bnb_ref[...]
        hb = jnp.maximum(y, 0.0) + hp_ref[...]
        agg = _agg_block(adj_ref, hb)
        hn = _dot(hb, ws_ref[...]) + _dot(agg, wn_ref[...]) + bs_ref[...]

        if with_stats:
            h_ref[...] = hb
            hn_ref[...] = hn
            acc_ref[0:1, :] += jnp.sum(hn, axis=0, keepdims=True)
            acc_ref[1:2, :] += jnp.sum(hn * hn, axis=0, keepdims=True)

            @pl.when(step == pl.num_programs(0) - 1)
            def _():
                souts_ref[...] = acc_ref[...]
        else:
            h_ref[...] = hn + hb

    big = pl.BlockSpec((_RB, _D), lambda i: (i, 0))
    row = pl.BlockSpec((1, _D), lambda i: (0, 0))
    stat = pl.BlockSpec((2, _D), lambda i: (0, 0))
    out_specs = [big]
    out_shapes = [jax.ShapeDtypeStruct((_N, _D), jnp.float32)]
    scratch = []
    if with_stats:
        out_specs += [big, stat]
        out_shapes += [jax.ShapeDtypeStruct((_N, _D), jnp.float32),
                       jax.ShapeDtypeStruct((2, _D), jnp.float32)]
        scratch.append(pltpu.VMEM((2, _D), jnp.float32))

    res = pl.pallas_call(
        body,
        grid=(_NSTEP,),
        in_specs=[
            big, big, stat, row, row,
            pl.BlockSpec((_GB, _NPG, _NPG), lambda i: (i, 0, 0)),
            pl.BlockSpec((_D, _D), lambda i: (0, 0)),
            pl.BlockSpec((_D, _D), lambda i: (0, 0)),
            row,
        ],
        out_specs=out_specs,
        out_shape=out_shapes,
        scratch_shapes=scratch,
    )(hn_prev, h_prev, sums_prev, bng, bnb, adj, ws, wn, bs)
    return res if with_stats else res[0]


def _pool_dsage(h, adj, wa_n, wa_s, ws0, wn0, ws1, wn1, ws2, wn2):
    def body(h_ref, adj_ref, wan_ref, was_ref, ws0_ref, wn0_ref, ws1_ref,
             wn1_ref, ws2_ref, wn2_ref, o_ref):
        hb = h_ref[...]
        was_term = _dot(hb, was_ref[...])
        t1s = []
        for g in range(_GB):
            a = adj_ref[g]
            adjn = a / jnp.maximum(jnp.sum(a, axis=1, keepdims=True), 1.0)
            t1s.append(_dot(adjn, hb[g * _NPG:(g + 1) * _NPG]))
        t1 = jnp.concatenate(t1s, axis=0)
        logits = _dot(t1, wan_ref[...]) + was_term
        m = jnp.max(logits, axis=1, keepdims=True)
        pexp = jnp.exp(logits - m)
        s = pexp / jnp.sum(pexp, axis=1, keepdims=True)

        xs = []
        apgs = []
        for g in range(_GB):
            sg = s[g * _NPG:(g + 1) * _NPG]
            hg = hb[g * _NPG:(g + 1) * _NPG]
            a = adj_ref[g]
            xs.append(_dot_t(sg, hg))
            adjp = _dot_t(sg, _dot(a, sg))
            apgs.append(adjp / jnp.maximum(
                jnp.sum(adjp, axis=1, keepdims=True), 1.0))
        x = jnp.concatenate(xs, axis=0)

        for i, (ws_ref, wn_ref) in enumerate(
                [(ws0_ref, wn0_ref), (ws1_ref, wn1_ref), (ws2_ref, wn2_ref)]):
            xwn = _dot(x, wn_ref[...])
            xws = _dot(x, ws_ref[...])
            xns = []
            for g in range(_GB):
                xns.append(_dot(apgs[g], xwn[g * _K:(g + 1) * _K])
                           + xws[g * _K:(g + 1) * _K])
            xn = jnp.concatenate(xns, axis=0)
            if i < 2:
                xn = jnp.maximum(xn, 0.0)
            x = xn + x
        o_ref[...] = x

    wspec = pl.BlockSpec((_D, _D), lambda i: (0, 0))
    return pl.pallas_call(
        body,
        grid=(_NSTEP,),
        in_specs=[
            pl.BlockSpec((_RB, _D), lambda i: (i, 0)),
            pl.BlockSpec((_GB, _NPG, _NPG), lambda i: (i, 0, 0)),
            pl.BlockSpec((_D, _K), lambda i: (0, 0)),
            pl.BlockSpec((_D, _K), lambda i: (0, 0)),
            wspec, wspec, wspec, wspec, wspec, wspec,
        ],
        out_specs=pl.BlockSpec((_KB, _D), lambda i: (i, 0)),
        out_shape=jax.ShapeDtypeStruct((_B * _K, _D), jnp.float32),
    )(h, adj, wa_n, wa_s, ws0, wn0, ws1, wn1, ws2, wn2)


def _readout(x, wp, bp):
    def body(x_ref, wp_ref, bp_ref, o_ref):
        r = jnp.sum(x_ref[...].reshape(_B, _K, _D), axis=1)
        o_ref[...] = _dot(r, wp_ref[...]) + bp_ref[...]

    return pl.pallas_call(
        body,
        in_specs=[
            pl.BlockSpec((_B * _K, _D), lambda: (0, 0)),
            pl.BlockSpec((_D, _NCLS), lambda: (0, 0)),
            pl.BlockSpec((1, _NCLS), lambda: (0, 0)),
        ],
        out_specs=pl.BlockSpec((_B, _NCLS), lambda: (0, 0)),
        out_shape=jax.ShapeDtypeStruct((_B, _NCLS), jnp.float32),
    )(x, wp, bp)


def kernel(h, edge_index, e, params):
    src = edge_index[0]
    dst = edge_index[1]

    adj_flat = _build_adj(src, dst)
    adj = adj_flat.reshape(_B, _NPG, _NPG)

    sage = params['sage']
    h0, hn0, sums0 = _sage0(h, adj, params['W_emb'],
                            params['b_emb'].reshape(1, _D),
                            sage[0]['Ws'], sage[0]['Wn'],
                            sage[0]['bs'].reshape(1, _D))
    h1, hn1, sums1 = _sage_next(
        hn0, h0, sums0, sage[0]['bn_g'].reshape(1, _D),
        sage[0]['bn_b'].reshape(1, _D), adj, sage[1]['Ws'], sage[1]['Wn'],
        sage[1]['bs'].reshape(1, _D), with_stats=True)
    h_final = _sage_next(
        hn1, h1, sums1, sage[1]['bn_g'].reshape(1, _D),
        sage[1]['bn_b'].reshape(1, _D), adj, sage[2]['Ws'], sage[2]['Wn'],
        sage[2]['bs'].reshape(1, _D), with_stats=False)

    d = params['dsage']
    x = _pool_dsage(h_final, adj, params['Wa_n'], params['Wa_s'],
                    d[0]['Ws'], d[0]['Wn'], d[1]['Ws'], d[1]['Wn'],
                    d[2]['Ws'], d[2]['Wn'])
    return _readout(x, params['W_pred'],
                    params['b_pred'].reshape(1, _NCLS))


# speedup vs baseline: 30.1718x; 1.2698x over previous
import functools

import jax
import jax.numpy as jnp
from jax import lax
from jax.experimental import pallas as pl
from jax.experimental.pallas import tpu as pltpu
from jax.experimental.pallas import tpu_sc as plsc

_N = 10000
_B = 100
_NPG = 100
_E = 160000
_D = 256
_K = 50
_NCLS = 2

_SC_CORES = 2
_SC_TILES = 16
_EPC = _E // _SC_CORES
_EPT = _EPC // _SC_TILES
_SEG = _B * _NPG * _NPG // _SC_CORES
_CH = 128
_NCH = (_EPT + _CH - 1) // _CH
_NG16 = _NCH * (_CH // 16)
_ZCH = 31248


def _build_adj(src, dst):
    mesh = plsc.VectorSubcoreMesh(core_axis_name="c", subcore_axis_name="s")

    @functools.partial(
        pl.kernel,
        out_type=jax.ShapeDtypeStruct((_B * _NPG * _NPG,), jnp.float32),
        mesh=mesh,
        scratch_types=[
            pltpu.VMEM((_NG16 * 16,), jnp.int32),
            pltpu.VMEM((_NG16 * 16,), jnp.int32),
            pltpu.VMEM((_NCH, _CH), jnp.int32),
            pltpu.VMEM((_NCH, _CH), jnp.float32),
            pltpu.VMEM((_ZCH,), jnp.float32),
            pltpu.VMEM_SHARED((_SEG,), jnp.float32),
            pltpu.SemaphoreType.DMA,
        ],
    )
    def adj_kernel(src_hbm, dst_hbm, out_hbm, src_v, dst_v, idx_v, val_v,
                   zero_v, acc_s, scat_sem):
        cid = lax.axis_index("c")
        tid = lax.axis_index("s")

        def zbody(i, _):
            zero_v[pl.ds(i * 16, 16)] = jnp.zeros((16,), jnp.float32)
            return 0

        lax.fori_loop(0, _ZCH // 16, zbody, 0)
        pltpu.sync_copy(zero_v, acc_s.at[pl.ds(tid * _ZCH, _ZCH)])

        @pl.when(tid == 0)
        def _():
            rem = _SEG - _SC_TILES * _ZCH
            pltpu.sync_copy(zero_v.at[pl.ds(0, rem)],
                            acc_s.at[pl.ds(_SC_TILES * _ZCH, rem)])

        base = cid * _EPC + tid * _EPT
        pltpu.sync_copy(src_hbm.at[pl.ds(base, _EPT)],
                        src_v.at[pl.ds(0, _EPT)])
        pltpu.sync_copy(dst_hbm.at[pl.ds(base, _EPT)],
                        dst_v.at[pl.ds(0, _EPT)])

        seg_base = cid * _SEG

        def cbody(g, _):
            sv = src_v[pl.ds(g * 16, 16)]
            dv = dst_v[pl.ds(g * 16, 16)]
            lane = g * 16 + lax.iota(jnp.int32, 16)
            valid = lane < _EPT
            idx = sv * _NPG + lax.rem(dv, _NPG) - seg_base
            idx = jnp.where(valid, idx, 0)
            val = jnp.where(valid, jnp.float32(1.0), jnp.float32(0.0))
            row = g // (_CH // 16)
            col = lax.rem(g, _CH // 16) * 16
            idx_v[row, pl.ds(col, 16)] = idx
            val_v[row, pl.ds(col, 16)] = val
            return 0

        lax.fori_loop(0, _NG16, cbody, 0)

        plsc.subcore_barrier()

        copies = [
            pltpu.async_copy(val_v.at[j], acc_s.at[idx_v.at[j]], scat_sem,
                             add=True)
            for j in range(_NCH)
        ]
        for c in copies:
            c.wait()

        plsc.subcore_barrier()

        out_base = cid * _SEG + tid * _ZCH
        pltpu.sync_copy(acc_s.at[pl.ds(tid * _ZCH, _ZCH)], zero_v)
        pltpu.sync_copy(zero_v, out_hbm.at[pl.ds(out_base, _ZCH)])

        @pl.when(tid == 0)
        def _():
            rem = _SEG - _SC_TILES * _ZCH
            pltpu.sync_copy(acc_s.at[pl.ds(_SC_TILES * _ZCH, rem)],
                            zero_v.at[pl.ds(0, rem)])
            pltpu.sync_copy(zero_v.at[pl.ds(0, rem)],
                            out_hbm.at[pl.ds(cid * _SEG + _SC_TILES * _ZCH,
                                             rem)])

    return adj_kernel(src, dst)


_GB = 20
_NSTEP = _B // _GB
_RB = _GB * _NPG


def _dot(a, b):
    return jnp.dot(a, b, preferred_element_type=jnp.float32)


def _dot_t(a, b):
    return lax.dot_general(a, b, (((0,), (0,)), ((), ())),
                           preferred_element_type=jnp.float32)


def _agg_block(adj_ref, hb, base):
    aggs = []
    for g in range(_GB):
        a = adj_ref[base + g]
        inv_indeg = 1.0 / jnp.maximum(jnp.sum(a, axis=0, keepdims=True), 1.0)
        aggs.append(_dot_t(a * inv_indeg, hb[g * _NPG:(g + 1) * _NPG]))
    return jnp.concatenate(aggs, axis=0)


def _net(h, adj, wemb, bemb, s_ws, s_wn, s_bs, s_bng, s_bnb,
         wa_n, wa_s, d_ws, d_wn, wp, bp):

    def body(h_ref, adj_ref, wemb_ref, bemb_ref, ws0_ref, ws1_ref, ws2_ref,
             wn0_ref, wn1_ref, wn2_ref, bs0_ref, bs1_ref, bs2_ref,
             bng0_ref, bng1_ref, bnb0_ref, bnb1_ref,
             wan_ref, was_ref, dws0_ref, dws1_ref, dws2_ref,
             dwn0_ref, dwn1_ref, dwn2_ref, wp_ref, bp_ref,
             o_ref, hc_ref, hn_ref, acc_ref, stats_ref):
        i = pl.program_id(0)
        phase = i // _NSTEP
        step = lax.rem(i, _NSTEP)
        rows = pl.ds(step * _RB, _RB)
        gbase = step * _GB

        @pl.when(i == 0)
        def _():
            acc_ref[...] = jnp.zeros_like(acc_ref)

        @pl.when((step == 0) & (phase > 0) & (phase < 3))
        def _():
            stats_ref[...] = acc_ref[...]
            acc_ref[...] = jnp.zeros_like(acc_ref)

        def accum(hn):
            acc_ref[0:1, :] += jnp.sum(hn, axis=0, keepdims=True)
            acc_ref[1:2, :] += jnp.sum(hn * hn, axis=0, keepdims=True)

        @pl.when(phase == 0)
        def _():
            h0 = _dot(h_ref[rows], wemb_ref[...]) + bemb_ref[...]
            agg = _agg_block(adj_ref, h0, gbase)
            hn = (_dot(h0, ws0_ref[...]) + _dot(agg, wn0_ref[...])
                  + bs0_ref[...])
            hc_ref[rows] = h0
            hn_ref[rows] = hn
            accum(hn)

        def sage_phase(ws_ref, wn_ref, bs_ref, bng_ref, bnb_ref, last):
            mu = stats_ref[0:1, :] * (1.0 / _N)
            var = stats_ref[1:2, :] * (1.0 / _N) - mu * mu
            scale = lax.rsqrt(var + 1e-5) * bng_ref[...]
            y = (hn_ref[rows] - mu) * scale + bnb_ref[...]
            hb = jnp.maximum(y, 0.0) + hc_ref[rows]
            agg = _agg_block(adj_ref, hb, gbase)
            hn = _dot(hb, ws_ref[...]) + _dot(agg, wn_ref[...]) + bs_ref[...]
            if last:
                hc_ref[rows] = hn + hb
            else:
                hc_ref[rows] = hb
                hn_ref[rows] = hn
                accum(hn)

        @pl.when(phase == 1)
        def _():
            sage_phase(ws1_ref, wn1_ref, bs1_ref, bng0_ref, bnb0_ref, False)

        @pl.when(phase == 2)
        def _():
            sage_phase(ws2_ref, wn2_ref, bs2_ref, bng1_ref, bnb1_ref, True)

        @pl.when(phase == 3)
        def _():
            hb = hc_ref[rows]
            was_term = _dot(hb, was_ref[...])
            t1s = []
            for g in range(_GB):
                a = adj_ref[gbase + g]
                adjn = a / jnp.maximum(jnp.sum(a, axis=1, keepdims=True), 1.0)
                t1s.append(_dot(adjn, hb[g * _NPG:(g + 1) * _NPG]))
            t1 = jnp.concatenate(t1s, axis=0)
            logits = _dot(t1, wan_ref[...]) + was_term
            m = jnp.max(logits, axis=1, keepdims=True)
            pexp = jnp.exp(logits - m)
            sm = pexp / jnp.sum(pexp, axis=1, keepdims=True)

            xs = []
            apgs = []
            for g in range(_GB):
                sg = sm[g * _NPG:(g + 1) * _NPG]
                hg = hb[g * _NPG:(g + 1) * _NPG]
                a = adj_ref[gbase + g]
                xs.append(_dot_t(sg, hg))
                adjp = _dot_t(sg, _dot(a, sg))
                apgs.append(adjp / jnp.maximum(
                    jnp.sum(adjp, axis=1, keepdims=True), 1.0))
            x = jnp.concatenate(xs, axis=0)

            for li, (dws_ref, dwn_ref) in enumerate(
                    [(dws0_ref, dwn0_ref), (dws1_ref, dwn1_ref),
                     (dws2_ref, dwn2_ref)]):
                xwn = _dot(x, dwn_ref[...])
                xws = _dot(x, dws_ref[...])
                xns = []
                for g in range(_GB):
                    xns.append(_dot(apgs[g], xwn[g * _K:(g + 1) * _K])
                               + xws[g * _K:(g + 1) * _K])
                xn = jnp.concatenate(xns, axis=0)
                if li < 2:
                    xn = jnp.maximum(xn, 0.0)
                x = xn + x

            r = jnp.sum(x.reshape(_GB, _K, _D), axis=1)
            o_ref[pl.ds(step * _GB, _GB)] = _dot(r, wp_ref[...]) + bp_ref[...]

    row = pl.BlockSpec((1, _D), lambda i: (0, 0))
    wsq = pl.BlockSpec((_D, _D), lambda i: (0, 0))
    wk = pl.BlockSpec((_D, _K), lambda i: (0, 0))
    return pl.pallas_call(
        body,
        grid=(4 * _NSTEP,),
        in_specs=[
            pl.BlockSpec((_N, _D), lambda i: (0, 0)),
            pl.BlockSpec((_B, _NPG, _NPG), lambda i: (0, 0, 0)),
            wsq, row,
            wsq, wsq, wsq, wsq, wsq, wsq, row, row, row,
            row, row, row, row,
            wk, wk, wsq, wsq, wsq, wsq, wsq, wsq,
            pl.BlockSpec((_D, _NCLS), lambda i: (0, 0)),
            pl.BlockSpec((1, _NCLS), lambda i: (0, 0)),
        ],
        out_specs=pl.BlockSpec((_B, _NCLS), lambda i: (0, 0)),
        out_shape=jax.ShapeDtypeStruct((_B, _NCLS), jnp.float32),
        scratch_shapes=[pltpu.VMEM((_N, _D), jnp.float32),
                        pltpu.VMEM((_N, _D), jnp.float32),
                        pltpu.VMEM((2, _D), jnp.float32),
                        pltpu.VMEM((2, _D), jnp.float32)],
    )(h, adj, wemb, bemb, *s_ws, *s_wn, *s_bs, *s_bng, *s_bnb,
      wa_n, wa_s, *d_ws, *d_wn, wp, bp)


def kernel(h, edge_index, e, params):
    src = edge_index[0]
    dst = edge_index[1]

    adj_flat = _build_adj(src, dst)
    adj = adj_flat.reshape(_B, _NPG, _NPG)

    sage = params['sage']
    d = params['dsage']
    return _net(
        h, adj, params['W_emb'], params['b_emb'].reshape(1, _D),
        [sage[i]['Ws'] for i in range(3)],
        [sage[i]['Wn'] for i in range(3)],
        [sage[i]['bs'].reshape(1, _D) for i in range(3)],
        [sage[i]['bn_g'].reshape(1, _D) for i in range(2)],
        [sage[i]['bn_b'].reshape(1, _D) for i in range(2)],
        params['Wa_n'], params['Wa_s'],
        [d[i]['Ws'] for i in range(3)],
        [d[i]['Wn'] for i in range(3)],
        params['W_pred'], params['b_pred'].reshape(1, _NCLS))
